# baseline (speedup 1.0000x reference)
import jax
import jax.numpy as jnp
from jax import lax
from jax.experimental import pallas as pl
from jax.experimental.pallas import tpu as pltpu
from jax.experimental.pallas import tpu_sc as plsc

N_USERS = 10000
N_ITEMS = 10000
N_PAD = 10240
N_NODES_P = 2 * N_PAD
EMBED = 128
N_EDGES = 320000
N_LAYERS_ = 3

N_CORES = 2
N_SUBCORES = 16
N_WORKERS = N_CORES * N_SUBCORES

CHUNK = 80
RING = 4
PREF = 2
EDGES_PER_WORKER = 20480
N_CHUNKS = EDGES_PER_WORKER // CHUNK
HALF_EDGES_P = N_SUBCORES * EDGES_PER_WORKER
EDGE_PAD = HALF_EDGES_P - N_EDGES
EDATA_WORDS = 2 * CHUNK

ROWS_PER_TILE = N_PAD // N_SUBCORES
STAGE_ROWS = 80
N_STAGE = ROWS_PER_TILE // STAGE_ROWS
N_GROUPS = EMBED // 16


def _mesh():
    return plsc.VectorSubcoreMesh(core_axis_name="c", subcore_axis_name="s")


def _fused_body(x0_hbm, edata_hbm, val_hbm, out_hbm, xall_hbm, acc, *rest):
    eb = rest[0:4]
    vb = rest[4:8]
    di = rest[8:12]
    si = rest[12:16]
    g = rest[16:20]
    stage = g[0]
    zb = g[1]
    es = rest[20:24]
    gs = rest[24:28]
    ss = rest[28:32]
    csem = rest[32]

    c = lax.axis_index("c")
    s = lax.axis_index("s")
    wid = c * N_SUBCORES + s

    def zero_row(r, _):
        for k in range(N_GROUPS):
            zb[r, pl.ds(k * 16, 16)] = jnp.zeros((16,), jnp.float32)
        return 0
    lax.fori_loop(0, STAGE_ROWS, zero_row, 0)
    for j in range(N_STAGE):
        pltpu.sync_copy(zb, acc.at[pl.ds(s * ROWS_PER_TILE + j * STAGE_ROWS,
                                         STAGE_ROWS)])
        base = wid * ROWS_PER_TILE + j * STAGE_ROWS
        pltpu.sync_copy(x0_hbm.at[pl.ds(base, STAGE_ROWS)], stage)
        pltpu.sync_copy(stage, xall_hbm.at[pl.ds(base, STAGE_ROWS)])
    plsc.subcore_barrier()
    pltpu.core_barrier(csem, core_axis_name="c")

    wchunk = (c * N_SUBCORES + s) * N_CHUNKS

    def eoff(t):
        return (wchunk + t) * EDATA_WORDS

    def voff(t):
        return (wchunk + t) * CHUNK

    def copy_didx(b):
        for k in range(CHUNK // 16):
            di[b][pl.ds(k * 16, 16)] = eb[b][pl.ds(k * 16, 16)]

    def copy_sidx(b, xoff):
        for k in range(CHUNK // 16):
            si[b][pl.ds(k * 16, 16)] = eb[b][pl.ds(CHUNK + k * 16, 16)] + xoff

    def prime(t, b, xoff):
        pltpu.sync_copy(edata_hbm.at[pl.ds(eoff(t), EDATA_WORDS)], eb[b])
        pltpu.sync_copy(val_hbm.at[pl.ds(voff(t), CHUNK)], vb[b])
        copy_didx(b)
        copy_sidx(b, xoff)
        pltpu.async_copy(xall_hbm.at[si[b]], g[b], gs[b])

    def scale(b):
        @plsc.parallel_loop(0, CHUNK // 16, unroll=2)
        def q_body(q):
            vv = vb[b][pl.ds(q * 16, 16)]
            for i in range(16):
                v = vv[i]
                r = q * 16 + i
                for k in range(N_GROUPS):
                    g[b][r, pl.ds(k * 16, 16)] = g[b][r, pl.ds(k * 16, 16)] * v

    def sub_body(p, j, xoff):
        t = p * RING + j
        b = j
        b2 = (j + PREF) % RING
        tp = t + PREF

        @pl.when(tp < N_CHUNKS)
        def _():
            pltpu.async_copy(edata_hbm.at[pl.ds(eoff(tp), EDATA_WORDS)],
                             eb[b2], es[b2])
            pltpu.async_copy(val_hbm.at[pl.ds(voff(tp), CHUNK)],
                             vb[b2], es[b2])

        pltpu.make_async_copy(xall_hbm.at[si[b]], g[b], gs[b]).wait()
        scale(b)
        pltpu.async_copy(g[b], acc.at[di[b]], ss[b], add=True)

        @pl.when(t >= PREF)
        def _():
            pltpu.make_async_copy(g[b2], acc.at[di[b2]], ss[b2]).wait()

        @pl.when(tp < N_CHUNKS)
        def _():
            pltpu.make_async_copy(edata_hbm.at[pl.ds(eoff(tp), EDATA_WORDS)],
                                  eb[b2], es[b2]).wait()
            pltpu.make_async_copy(val_hbm.at[pl.ds(voff(tp), CHUNK)],
                                  vb[b2], es[b2]).wait()
            copy_didx(b2)
            copy_sidx(b2, xoff)
            pltpu.async_copy(xall_hbm.at[si[b2]], g[b2], gs[b2])

    def layer_step(l, _):
        xoff = l * N_NODES_P
        prime(0, 0, xoff)
        prime(1, 1, xoff)

        def p_body(p, _):
            for j in range(RING):
                sub_body(p, j, xoff)
            return 0
        lax.fori_loop(0, N_CHUNKS // RING, p_body, 0)

        for b in ((N_CHUNKS - PREF) % RING, (N_CHUNKS - 1) % RING):
            pltpu.make_async_copy(g[b], acc.at[di[b]], ss[b]).wait()
        plsc.subcore_barrier()

        def zrow(r, _):
            for k in range(N_GROUPS):
                zb[r, pl.ds(k * 16, 16)] = jnp.zeros((16,), jnp.float32)
            return 0
        lax.fori_loop(0, STAGE_ROWS, zrow, 0)
        obase = (l + 1) * N_NODES_P + c * N_PAD + s * ROWS_PER_TILE
        for j in range(N_STAGE):
            sl = pl.ds(s * ROWS_PER_TILE + j * STAGE_ROWS, STAGE_ROWS)
            pltpu.async_copy(acc.at[sl],
                             xall_hbm.at[pl.ds(obase + j * STAGE_ROWS,
                                               STAGE_ROWS)], gs[0])
        for j in range(N_STAGE):
            sl = pl.ds(s * ROWS_PER_TILE + j * STAGE_ROWS, STAGE_ROWS)
            pltpu.make_async_copy(acc.at[sl],
                                  xall_hbm.at[pl.ds(obase + j * STAGE_ROWS,
                                                    STAGE_ROWS)], gs[0]).wait()
        for j in range(N_STAGE):
            sl = pl.ds(s * ROWS_PER_TILE + j * STAGE_ROWS, STAGE_ROWS)
            pltpu.async_copy(zb, acc.at[sl], gs[1])
        for j in range(N_STAGE):
            sl = pl.ds(s * ROWS_PER_TILE + j * STAGE_ROWS, STAGE_ROWS)
            pltpu.make_async_copy(zb, acc.at[sl], gs[1]).wait()
        plsc.subcore_barrier()
        pltpu.core_barrier(csem, core_axis_name="c")
        return 0

    lax.fori_loop(0, N_LAYERS_, layer_step, 0)

    mbase = c * N_PAD + s * ROWS_PER_TILE

    def mean_chunk(j, _):
        base = mbase + j * STAGE_ROWS
        for l in range(4):
            pltpu.async_copy(xall_hbm.at[pl.ds(l * N_NODES_P + base,
                                               STAGE_ROWS)], g[l], gs[l])
        for l in range(4):
            pltpu.make_async_copy(xall_hbm.at[pl.ds(l * N_NODES_P + base,
                                                    STAGE_ROWS)], g[l],
                                  gs[l]).wait()

        def mean_row(r, _):
            for k in range(N_GROUPS):
                sl = pl.ds(k * 16, 16)
                g[0][r, sl] = (g[0][r, sl] + g[1][r, sl]
                               + g[2][r, sl] + g[3][r, sl]) * 0.25
            return 0
        lax.fori_loop(0, STAGE_ROWS, mean_row, 0)
        pltpu.sync_copy(g[0], out_hbm.at[pl.ds(base, STAGE_ROWS)])
        return 0

    lax.fori_loop(0, N_STAGE, mean_chunk, 0)


@jax.jit
def _propagate(x0, edata, val_all):
    fused = pl.kernel(
        _fused_body,
        out_type=(
            jax.ShapeDtypeStruct((N_NODES_P, EMBED), jnp.float32),
            jax.ShapeDtypeStruct(((N_LAYERS_ + 1) * N_NODES_P, EMBED),
                                 jnp.float32),
        ),
        mesh=_mesh(),
        scratch_types=(
            [pltpu.VMEM_SHARED((N_PAD, EMBED), jnp.float32)]
            + [pltpu.VMEM((EDATA_WORDS,), jnp.int32)] * RING
            + [pltpu.VMEM((CHUNK,), jnp.float32)] * RING
            + [pltpu.VMEM((CHUNK,), jnp.int32)] * RING
            + [pltpu.VMEM((CHUNK,), jnp.int32)] * RING
            + [pltpu.VMEM((CHUNK, EMBED), jnp.float32)] * RING
            + [pltpu.SemaphoreType.DMA] * (3 * RING)
            + [pltpu.SemaphoreType.REGULAR]
        ),
    )
    out, _ = fused(x0, edata, val_all)
    return out


def kernel(user_preference, item_preference, edge_values, edge_index):
    rows = edge_index[0].astype(jnp.int32)
    cols = edge_index[1].astype(jnp.int32)
    par = jnp.arange(EDGE_PAD, dtype=jnp.int32) % N_USERS
    zval = jnp.zeros((EDGE_PAD,), jnp.float32)
    dst_all = jnp.concatenate([rows, par, cols, par])
    src_all = jnp.concatenate([cols + N_PAD, par + N_PAD, rows, par])
    val_all = jnp.concatenate([edge_values, zval, edge_values, zval])
    edata = jnp.stack([dst_all.reshape(-1, CHUNK),
                       src_all.reshape(-1, CHUNK)], axis=1).reshape(-1)

    zpad = jnp.zeros((N_PAD - N_USERS, EMBED), jnp.float32)
    x0 = jnp.concatenate([user_preference, zpad, item_preference, zpad],
                         axis=0)
    out = _propagate(x0, edata, val_all)
    return out[:N_USERS], out[N_PAD:N_PAD + N_ITEMS]


# speedup vs baseline: 1.1476x; 1.1476x over previous
import jax
import jax.numpy as jnp
from jax import lax
from jax.experimental import pallas as pl
from jax.experimental.pallas import tpu as pltpu
from jax.experimental.pallas import tpu_sc as plsc

N_USERS = 10000
N_ITEMS = 10000
N_PAD = 10240
N_NODES_P = 2 * N_PAD
EMBED = 128
N_EDGES = 320000
N_LAYERS_ = 3

N_CORES = 2
N_SUBCORES = 16
N_WORKERS = N_CORES * N_SUBCORES

CHUNK = 80
RING = 4
PREF = 2
EDGES_PER_WORKER = 20480
N_CHUNKS = EDGES_PER_WORKER // CHUNK
HALF_EDGES_P = N_SUBCORES * EDGES_PER_WORKER
EDGE_PAD = HALF_EDGES_P - N_EDGES
EDATA_WORDS = 2 * CHUNK

ROWS_PER_TILE = N_PAD // N_SUBCORES
STAGE_ROWS = 80
N_STAGE = ROWS_PER_TILE // STAGE_ROWS
N_GROUPS = EMBED // 16


def _mesh():
    return plsc.VectorSubcoreMesh(core_axis_name="c", subcore_axis_name="s")


def _fused_body(x0_hbm, edata_hbm, val_hbm, out_hbm, xall_hbm, acc, *rest):
    eb = rest[0:4]
    vb = rest[4:8]
    di = rest[8:12]
    si = rest[12:16]
    g = rest[16:20]
    stage = g[0]
    zb = g[1]
    es = rest[20:24]
    gs = rest[24:28]
    ss = rest[28:32]
    csem = rest[32]

    c = lax.axis_index("c")
    s = lax.axis_index("s")
    wid = c * N_SUBCORES + s

    def zero_row(r, _):
        for k in range(N_GROUPS):
            zb[r, pl.ds(k * 16, 16)] = jnp.zeros((16,), jnp.float32)
        return 0
    lax.fori_loop(0, STAGE_ROWS, zero_row, 0)
    for j in range(N_STAGE):
        pltpu.async_copy(zb, acc.at[pl.ds(s * ROWS_PER_TILE + j * STAGE_ROWS,
                                          STAGE_ROWS)], gs[1])
    for j in range(N_STAGE):
        base = wid * ROWS_PER_TILE + j * STAGE_ROWS
        pltpu.sync_copy(x0_hbm.at[pl.ds(base, STAGE_ROWS)], stage)
        pltpu.sync_copy(stage, xall_hbm.at[pl.ds(base, STAGE_ROWS)])
    for j in range(N_STAGE):
        pltpu.make_async_copy(zb, acc.at[pl.ds(s * ROWS_PER_TILE
                                               + j * STAGE_ROWS,
                                               STAGE_ROWS)], gs[1]).wait()
    plsc.subcore_barrier()
    pltpu.core_barrier(csem, core_axis_name="c")

    wchunk = (c * N_SUBCORES + s) * N_CHUNKS

    def eoff(t):
        return (wchunk + t) * EDATA_WORDS

    def voff(t):
        return (wchunk + t) * CHUNK

    def copy_didx(b):
        for k in range(CHUNK // 16):
            di[b][pl.ds(k * 16, 16)] = eb[b][pl.ds(k * 16, 16)]

    def copy_sidx(b, xoff):
        for k in range(CHUNK // 16):
            si[b][pl.ds(k * 16, 16)] = eb[b][pl.ds(CHUNK + k * 16, 16)] + xoff

    def prime(t, b, xoff):
        pltpu.sync_copy(edata_hbm.at[pl.ds(eoff(t), EDATA_WORDS)], eb[b])
        pltpu.sync_copy(val_hbm.at[pl.ds(voff(t), CHUNK)], vb[b])
        copy_didx(b)
        copy_sidx(b, xoff)
        pltpu.async_copy(xall_hbm.at[si[b]], g[b], gs[b])

    def scale(b):
        def q_body(q, _):
            vv = vb[b][pl.ds(q * 16, 16)]
            for i in range(16):
                v = vv[i]
                r = q * 16 + i
                for k in range(N_GROUPS):
                    g[b][r, pl.ds(k * 16, 16)] = g[b][r, pl.ds(k * 16, 16)] * v
            return 0
        lax.fori_loop(0, CHUNK // 16, q_body, 0)

    def sub_body(p, j, xoff):
        t = p * RING + j
        b = j
        b2 = (j + PREF) % RING
        tp = t + PREF

        @pl.when(tp < N_CHUNKS)
        def _():
            pltpu.async_copy(edata_hbm.at[pl.ds(eoff(tp), EDATA_WORDS)],
                             eb[b2], es[b2])
            pltpu.async_copy(val_hbm.at[pl.ds(voff(tp), CHUNK)],
                             vb[b2], es[b2])

        pltpu.make_async_copy(xall_hbm.at[si[b]], g[b], gs[b]).wait()
        scale(b)
        pltpu.async_copy(g[b], acc.at[di[b]], ss[b], add=True)

        @pl.when(t >= PREF)
        def _():
            pltpu.make_async_copy(g[b2], acc.at[di[b2]], ss[b2]).wait()

        @pl.when(tp < N_CHUNKS)
        def _():
            pltpu.make_async_copy(edata_hbm.at[pl.ds(eoff(tp), EDATA_WORDS)],
                                  eb[b2], es[b2]).wait()
            pltpu.make_async_copy(val_hbm.at[pl.ds(voff(tp), CHUNK)],
                                  vb[b2], es[b2]).wait()
            copy_didx(b2)
            copy_sidx(b2, xoff)
            pltpu.async_copy(xall_hbm.at[si[b2]], g[b2], gs[b2])

    def layer_step(l, _):
        xoff = l * N_NODES_P
        prime(0, 0, xoff)
        prime(1, 1, xoff)

        def p_body(p, _):
            for j in range(RING):
                sub_body(p, j, xoff)
            return 0
        lax.fori_loop(0, N_CHUNKS // RING, p_body, 0)

        for b in ((N_CHUNKS - PREF) % RING, (N_CHUNKS - 1) % RING):
            pltpu.make_async_copy(g[b], acc.at[di[b]], ss[b]).wait()
        plsc.subcore_barrier()

        @pl.when(l < N_LAYERS_ - 1)
        def _():
            def zrow(r, _):
                for k in range(N_GROUPS):
                    zb[r, pl.ds(k * 16, 16)] = jnp.zeros((16,), jnp.float32)
                return 0
            lax.fori_loop(0, STAGE_ROWS, zrow, 0)
            obase = (l + 1) * N_NODES_P + c * N_PAD + s * ROWS_PER_TILE
            for j in range(N_STAGE):
                sl = pl.ds(s * ROWS_PER_TILE + j * STAGE_ROWS, STAGE_ROWS)
                pltpu.async_copy(acc.at[sl],
                                 xall_hbm.at[pl.ds(obase + j * STAGE_ROWS,
                                                   STAGE_ROWS)], gs[0])
            for j in range(N_STAGE):
                sl = pl.ds(s * ROWS_PER_TILE + j * STAGE_ROWS, STAGE_ROWS)
                pltpu.make_async_copy(
                    acc.at[sl],
                    xall_hbm.at[pl.ds(obase + j * STAGE_ROWS,
                                      STAGE_ROWS)], gs[0]).wait()
            for j in range(N_STAGE):
                sl = pl.ds(s * ROWS_PER_TILE + j * STAGE_ROWS, STAGE_ROWS)
                pltpu.async_copy(zb, acc.at[sl], gs[1])
            for j in range(N_STAGE):
                sl = pl.ds(s * ROWS_PER_TILE + j * STAGE_ROWS, STAGE_ROWS)
                pltpu.make_async_copy(zb, acc.at[sl], gs[1]).wait()
        plsc.subcore_barrier()
        pltpu.core_barrier(csem, core_axis_name="c")
        return 0

    lax.fori_loop(0, N_LAYERS_, layer_step, 0)

    mbase = c * N_PAD + s * ROWS_PER_TILE

    def mean_chunk(j, _):
        base = mbase + j * STAGE_ROWS
        accsl = pl.ds(s * ROWS_PER_TILE + j * STAGE_ROWS, STAGE_ROWS)
        for l in range(3):
            pltpu.async_copy(xall_hbm.at[pl.ds(l * N_NODES_P + base,
                                               STAGE_ROWS)], g[l], gs[l])
        pltpu.async_copy(acc.at[accsl], g[3], gs[3])
        for l in range(3):
            pltpu.make_async_copy(xall_hbm.at[pl.ds(l * N_NODES_P + base,
                                                    STAGE_ROWS)], g[l],
                                  gs[l]).wait()
        pltpu.make_async_copy(acc.at[accsl], g[3], gs[3]).wait()

        def mean_row(r, _):
            for k in range(N_GROUPS):
                sl = pl.ds(k * 16, 16)
                g[0][r, sl] = (g[0][r, sl] + g[1][r, sl]
                               + g[2][r, sl] + g[3][r, sl]) * 0.25
            return 0
        lax.fori_loop(0, STAGE_ROWS, mean_row, 0)
        pltpu.sync_copy(g[0], out_hbm.at[pl.ds(base, STAGE_ROWS)])
        return 0

    lax.fori_loop(0, N_STAGE, mean_chunk, 0)


@jax.jit
def _propagate(x0, edata, val_all):
    fused = pl.kernel(
        _fused_body,
        out_type=(
            jax.ShapeDtypeStruct((N_NODES_P, EMBED), jnp.float32),
            jax.ShapeDtypeStruct(((N_LAYERS_ + 1) * N_NODES_P, EMBED),
                                 jnp.float32),
        ),
        mesh=_mesh(),
        scratch_types=(
            [pltpu.VMEM_SHARED((N_PAD, EMBED), jnp.float32)]
            + [pltpu.VMEM((EDATA_WORDS,), jnp.int32)] * RING
            + [pltpu.VMEM((CHUNK,), jnp.float32)] * RING
            + [pltpu.VMEM((CHUNK,), jnp.int32)] * RING
            + [pltpu.VMEM((CHUNK,), jnp.int32)] * RING
            + [pltpu.VMEM((CHUNK, EMBED), jnp.float32)] * RING
            + [pltpu.SemaphoreType.DMA] * (3 * RING)
            + [pltpu.SemaphoreType.REGULAR]
        ),
    )
    out, _ = fused(x0, edata, val_all)
    return out


def kernel(user_preference, item_preference, edge_values, edge_index):
    rows = edge_index[0].astype(jnp.int32)
    cols = edge_index[1].astype(jnp.int32)
    par = jnp.arange(EDGE_PAD, dtype=jnp.int32) % N_USERS
    zval = jnp.zeros((EDGE_PAD,), jnp.float32)
    dst_all = jnp.concatenate([rows, par, cols, par])
    src_all = jnp.concatenate([cols + N_PAD, par + N_PAD, rows, par])
    val_all = jnp.concatenate([edge_values, zval, edge_values, zval])
    edata = jnp.stack([dst_all.reshape(-1, CHUNK),
                       src_all.reshape(-1, CHUNK)], axis=1).reshape(-1)

    zpad = jnp.zeros((N_PAD - N_USERS, EMBED), jnp.float32)
    x0 = jnp.concatenate([user_preference, zpad, item_preference, zpad],
                         axis=0)
    out = _propagate(x0, edata, val_all)
    return out[:N_USERS], out[N_PAD:N_PAD + N_ITEMS]
